# baseline (speedup 1.0000x reference)
import functools

import jax
import jax.numpy as jnp
from jax import lax
from jax.experimental import pallas as pl
from jax.experimental.pallas import tpu as pltpu
from jax.experimental.pallas import tpu_sc as plsc

_NC = 2
_NS = 16
_NW = _NC * _NS
_CHUNK = 128
_RING = 3
_L = 16


def _sc_gather_scale(table, inp, scale):
    B0, S = inp.shape
    V, D = table.shape
    B = B0 * S
    assert B % (_NW * _CHUNK) == 0 and D % _L == 0
    assert B0 % _NW == 0 and S % 8 == 0 and S >= _L
    copy_offs = list(range(0, S - _L + 1, _L))
    if S % _L:
        copy_offs.append(S - _L)
    rows_per_w = B0 // _NW
    b_per_w = B // _NW
    n_chunk = b_per_w // _CHUNK
    scale = float(scale)
    mesh = plsc.VectorSubcoreMesh(core_axis_name="c", subcore_axis_name="s")

    @functools.partial(
        pl.kernel,
        mesh=mesh,
        out_type=jax.ShapeDtypeStruct((B, D), table.dtype),
        scratch_types=[
            pltpu.VMEM((rows_per_w, S), jnp.int32),
            pltpu.VMEM((b_per_w,), jnp.int32),
            pltpu.VMEM((_RING, _CHUNK, D), table.dtype),
            pltpu.VMEM((_RING, _CHUNK, D), table.dtype),
            pltpu.SemaphoreType.DMA,
            pltpu.SemaphoreType.DMA,
            pltpu.SemaphoreType.DMA,
            pltpu.SemaphoreType.DMA,
            pltpu.SemaphoreType.DMA,
            pltpu.SemaphoreType.DMA,
        ],
    )
    def k(table_hbm, idx_hbm, out_hbm, idx2, idx_v, gbuf, sbuf,
          gs0, gs1, gs2, ss0, ss1, ss2):
        gsems = (gs0, gs1, gs2)
        ssems = (ss0, ss1, ss2)
        wid = lax.axis_index("s") * _NC + lax.axis_index("c")
        base = wid * b_per_w
        pltpu.sync_copy(idx_hbm.at[pl.ds(wid * rows_per_w, rows_per_w)], idx2)

        @plsc.parallel_loop(0, rows_per_w, unroll=2)
        def _(r):
            for off in copy_offs:
                idx_v[pl.ds(r * S + off, _L)] = idx2[r, pl.ds(off, _L)]

        def gather(i, b):
            pltpu.async_copy(
                table_hbm.at[idx_v.at[pl.ds(i * _CHUNK, _CHUNK)]],
                gbuf.at[b], gsems[b])

        def drain(ref, sem):
            pltpu.make_async_copy(
                table_hbm.at[idx_v.at[pl.ds(0, _CHUNK)]], ref, sem).wait()

        for b in range(_RING):
            gather(b, b)

        n_iter = -(-n_chunk // _RING) * _RING

        @pl.loop(0, n_iter, step=_RING)
        def _(g):
            for b in range(_RING):
                i = g + b

                @pl.when(i < n_chunk)
                def _():
                    drain(gbuf.at[b], gsems[b])

                    @pl.when(i >= _RING)
                    def _():
                        drain(sbuf.at[b], ssems[b])

                    @plsc.parallel_loop(0, _CHUNK, unroll=4)
                    def _(r):
                        for j in range(D // _L):
                            sl = pl.ds(j * _L, _L)
                            sbuf[b, r, sl] = gbuf[b, r, sl] * scale

                    @pl.when(i + _RING < n_chunk)
                    def _():
                        gather(i + _RING, b)
                    pltpu.async_copy(
                        sbuf.at[b],
                        out_hbm.at[pl.ds(base + i * _CHUNK, _CHUNK)],
                        ssems[b])

        for b in range(_RING):
            drain(sbuf.at[b], ssems[b])

    return k(table, inp)


def kernel(inp, W):
    B0, S = inp.shape
    V, D = W.shape
    if inp.dtype != jnp.int32:
        inp = inp.astype(jnp.int32)
    out = _sc_gather_scale(W, inp, float(D) ** 0.5)
    return out.reshape(B0, S, D)


# speedup vs baseline: 1.0048x; 1.0048x over previous
import functools

import jax
import jax.numpy as jnp
from jax import lax
from jax.experimental import pallas as pl
from jax.experimental.pallas import tpu as pltpu
from jax.experimental.pallas import tpu_sc as plsc

_NC = 2
_NS = 16
_NW = _NC * _NS
_CHUNK = 128
_RING = 3
_L = 16


def _sc_gather_scale(table, idx, scale):
    (B,) = idx.shape
    V, D = table.shape
    assert B % (_NW * _CHUNK) == 0 and D % _L == 0
    b_per_w = B // _NW
    n_chunk = b_per_w // _CHUNK
    scale = float(scale)
    mesh = plsc.VectorSubcoreMesh(core_axis_name="c", subcore_axis_name="s")

    @functools.partial(
        pl.kernel,
        mesh=mesh,
        out_type=jax.ShapeDtypeStruct((B, D), table.dtype),
        scratch_types=[
            pltpu.VMEM((b_per_w,), jnp.int32),
            pltpu.VMEM((_RING, _CHUNK, D), table.dtype),
            pltpu.VMEM((_RING, _CHUNK, D), table.dtype),
            pltpu.SemaphoreType.DMA,
            pltpu.SemaphoreType.DMA,
            pltpu.SemaphoreType.DMA,
            pltpu.SemaphoreType.DMA,
            pltpu.SemaphoreType.DMA,
            pltpu.SemaphoreType.DMA,
        ],
    )
    def k(table_hbm, idx_hbm, out_hbm, idx_v, gbuf, sbuf,
          gs0, gs1, gs2, ss0, ss1, ss2):
        gsems = (gs0, gs1, gs2)
        ssems = (ss0, ss1, ss2)
        wid = lax.axis_index("s") * _NC + lax.axis_index("c")
        base = wid * b_per_w
        pltpu.sync_copy(idx_hbm.at[pl.ds(base, b_per_w)], idx_v)

        def gather(i, b):
            pltpu.async_copy(
                table_hbm.at[idx_v.at[pl.ds(i * _CHUNK, _CHUNK)]],
                gbuf.at[b], gsems[b])

        def drain(ref, sem):
            pltpu.make_async_copy(
                table_hbm.at[idx_v.at[pl.ds(0, _CHUNK)]], ref, sem).wait()

        for b in range(_RING):
            gather(b, b)

        n_iter = -(-n_chunk // _RING) * _RING

        @pl.loop(0, n_iter, step=_RING)
        def _(g):
            for b in range(_RING):
                i = g + b

                @pl.when(i < n_chunk)
                def _():
                    drain(gbuf.at[b], gsems[b])

                    @pl.when(i >= _RING)
                    def _():
                        drain(sbuf.at[b], ssems[b])

                    @plsc.parallel_loop(0, _CHUNK, unroll=4)
                    def _(r):
                        for j in range(D // _L):
                            sl = pl.ds(j * _L, _L)
                            sbuf[b, r, sl] = gbuf[b, r, sl] * scale

                    @pl.when(i + _RING < n_chunk)
                    def _():
                        gather(i + _RING, b)
                    pltpu.async_copy(
                        sbuf.at[b],
                        out_hbm.at[pl.ds(base + i * _CHUNK, _CHUNK)],
                        ssems[b])

        for b in range(_RING):
            drain(sbuf.at[b], ssems[b])

    return k(table, idx)


def kernel(inp, W):
    B0, S = inp.shape
    V, D = W.shape
    idx = inp.reshape(B0 * S).astype(jnp.int32)
    out = _sc_gather_scale(W, idx, float(D) ** 0.5)
    return out.reshape(B0, S, D)


# speedup vs baseline: 1.0069x; 1.0021x over previous
import functools

import jax
import jax.numpy as jnp
from jax import lax
from jax.experimental import pallas as pl
from jax.experimental.pallas import tpu as pltpu
from jax.experimental.pallas import tpu_sc as plsc

_NC = 2
_NS = 16
_NW = _NC * _NS
_CHUNK = 128
_GRING = 4
_SRING = 2
_L = 16


def _sc_gather_scale(table, idx, scale):
    (B,) = idx.shape
    V, D = table.shape
    assert B % (_NW * _CHUNK) == 0 and D % _L == 0
    b_per_w = B // _NW
    n_chunk = b_per_w // _CHUNK
    scale = float(scale)
    mesh = plsc.VectorSubcoreMesh(core_axis_name="c", subcore_axis_name="s")

    @functools.partial(
        pl.kernel,
        mesh=mesh,
        out_type=jax.ShapeDtypeStruct((B, D), table.dtype),
        scratch_types=[
            pltpu.VMEM((b_per_w,), jnp.int32),
            pltpu.VMEM((_GRING, _CHUNK, D), table.dtype),
            pltpu.VMEM((_SRING, _CHUNK, D), table.dtype),
            pltpu.SemaphoreType.DMA,
            pltpu.SemaphoreType.DMA,
            pltpu.SemaphoreType.DMA,
            pltpu.SemaphoreType.DMA,
            pltpu.SemaphoreType.DMA,
            pltpu.SemaphoreType.DMA,
        ],
    )
    def k(table_hbm, idx_hbm, out_hbm, idx_v, gbuf, sbuf,
          gs0, gs1, gs2, gs3, ss0, ss1):
        gsems = (gs0, gs1, gs2, gs3)
        ssems = (ss0, ss1)
        wid = lax.axis_index("s") * _NC + lax.axis_index("c")
        base = wid * b_per_w
        pltpu.sync_copy(idx_hbm.at[pl.ds(base, b_per_w)], idx_v)

        def gather(i, bg):
            pltpu.async_copy(
                table_hbm.at[idx_v.at[pl.ds(i * _CHUNK, _CHUNK)]],
                gbuf.at[bg], gsems[bg])

        def drain(ref, sem):
            pltpu.make_async_copy(
                table_hbm.at[idx_v.at[pl.ds(0, _CHUNK)]], ref, sem).wait()

        for b in range(_GRING):
            gather(b, b)

        n_iter = -(-n_chunk // _GRING) * _GRING

        @pl.loop(0, n_iter, step=_GRING)
        def _(g):
            for b in range(_GRING):
                i = g + b
                bs = b % _SRING

                @pl.when(i < n_chunk)
                def _():
                    drain(gbuf.at[b], gsems[b])

                    @pl.when(i >= _SRING)
                    def _():
                        drain(sbuf.at[bs], ssems[bs])

                    @plsc.parallel_loop(0, _CHUNK, unroll=4)
                    def _(r):
                        for j in range(D // _L):
                            sl = pl.ds(j * _L, _L)
                            sbuf[bs, r, sl] = gbuf[b, r, sl] * scale

                    @pl.when(i + _GRING < n_chunk)
                    def _():
                        gather(i + _GRING, b)
                    pltpu.async_copy(
                        sbuf.at[bs],
                        out_hbm.at[pl.ds(base + i * _CHUNK, _CHUNK)],
                        ssems[bs])

        for b in range(_SRING):
            drain(sbuf.at[b], ssems[b])

    return k(table, idx)


def kernel(inp, W):
    B0, S = inp.shape
    V, D = W.shape
    idx = inp.reshape(B0 * S).astype(jnp.int32)
    out = _sc_gather_scale(W, idx, float(D) ** 0.5)
    return out.reshape(B0, S, D)
